# baseline (device time: 50879 ns/iter reference)
import jax
import jax.numpy as jnp
from jax import lax
from jax.experimental import pallas as pl
from jax.experimental.pallas import tpu as pltpu

N_DEV = 16
SQ = 256
CHUNK = SQ // N_DEV
HALF = SQ // 2
SKV_LOCAL = 4096
HQ = 8
HKV = 2
DH = 128
D = 1024
SCALE = 0.08838834764831843
LOG2E = 1.4426950408889634

COMM_W = D + 128


def kernel(x, Wq, Wo, K_ext, V_ext):
    x2 = x.reshape(SQ, D)
    k3 = K_ext.reshape(SKV_LOCAL, HKV, DH)
    v3 = V_ext.reshape(SKV_LOCAL, HKV, DH)

    def body(x_ref, wq_ref, wo_ref, k_ref, v_ref, out_ref,
             pack_ref, p1_ref, f_ref, g2_ref, kv_ref,
             s1_sem, r1_sem, s2_sem, r2_sem, cp_sem):
        my = lax.axis_index("i")

        copies = []
        for t, src in enumerate((k_ref, v_ref)):
            for kvh in range(HKV):
                c = pltpu.make_async_copy(
                    src.at[:, kvh, :],
                    kv_ref.at[:, (t * HKV + kvh) * DH:
                              (t * HKV + kvh + 1) * DH],
                    cp_sem,
                )
                c.start()
                copies.append(c)

        barrier_sem = pltpu.get_barrier_semaphore()
        for d in range(1, N_DEV):
            peer = lax.rem(my + d, N_DEV)
            pl.semaphore_signal(
                barrier_sem, inc=1,
                device_id=(peer,), device_id_type=pl.DeviceIdType.MESH,
            )

        dests = []
        rdma1 = []
        for d in range(1, N_DEV):
            dest = lax.rem(my + d, N_DEV)
            dests.append(dest)
            rdma1.append(pltpu.make_async_remote_copy(
                src_ref=pack_ref.at[pl.ds(dest * CHUNK, CHUNK), :],
                dst_ref=p1_ref.at[d],
                send_sem=s1_sem,
                recv_sem=r1_sem,
                device_id=(dest,),
                device_id_type=pl.DeviceIdType.MESH,
            ))

        qb = (jnp.dot(x_ref[...].astype(jnp.bfloat16),
                      wq_ref[...].astype(jnp.bfloat16),
                      preferred_element_type=jnp.float32)
              * (SCALE * LOG2E)).astype(jnp.bfloat16)
        for c in copies:
            c.wait()
        kvb = kv_ref[...].astype(jnp.bfloat16)
        kb = kvb[:, :HKV * DH]
        vb = kvb[:, HKV * DH:]

        for half in range(2):
            r0 = half * HALF
            for h in range(HQ):
                kvh = h // (HQ // HKV)
                q_h = qb[r0:r0 + HALF, h * DH:(h + 1) * DH]
                k_h = kb[:, kvh * DH:(kvh + 1) * DH]
                v_h = vb[:, kvh * DH:(kvh + 1) * DH]
                s = lax.dot_general(
                    q_h, k_h, (((1,), (1,)), ((), ())),
                    preferred_element_type=jnp.float32,
                )
                p = jnp.exp2(s)
                l_h = jnp.sum(p, axis=1, keepdims=True)
                o_h = jnp.dot(p.astype(jnp.bfloat16), v_h,
                              preferred_element_type=jnp.float32)
                sl = pl.ds(r0, HALF)
                pack_ref[sl, h * DH:(h + 1) * DH] = o_h.astype(jnp.bfloat16)
                pack_ref[sl, D + h:D + h + 1] = l_h.astype(jnp.bfloat16)

            if half == 0:
                pl.semaphore_wait(barrier_sem, N_DEV - 1)
            for i, dest in enumerate(dests):
                row = dest * CHUNK
                in_half = jnp.logical_and(row >= r0, row < r0 + HALF)
                @pl.when(in_half)
                def _(r=rdma1[i]):
                    r.start()

        p1_ref[0] = pack_ref[pl.ds(my * CHUNK, CHUNK), :]

        for r in rdma1:
            r.wait_recv()

        arr = p1_ref[...].astype(jnp.float32)
        acc = jnp.sum(arr, axis=0)
        o_heads = []
        for h in range(HQ):
            l_c = acc[:, D + h:D + h + 1]
            o_c = acc[:, h * DH:(h + 1) * DH]
            o_heads.append(o_c / l_c)
        o_n = jnp.concatenate(o_heads, axis=1)

        final = jnp.dot(o_n, wo_ref[...],
                        preferred_element_type=jnp.float32)
        f_ref[...] = final.astype(jnp.bfloat16)
        g2_ref[pl.ds(my * CHUNK, CHUNK), :] = f_ref[...]

        rdma2 = []
        for d in range(1, N_DEV):
            dest = lax.rem(my + d, N_DEV)
            r = pltpu.make_async_remote_copy(
                src_ref=f_ref,
                dst_ref=g2_ref.at[pl.ds(my * CHUNK, CHUNK), :],
                send_sem=s2_sem,
                recv_sem=r2_sem,
                device_id=(dest,),
                device_id_type=pl.DeviceIdType.MESH,
            )
            r.start()
            rdma2.append(r)

        for r in rdma1:
            r.wait_send()
        for r in rdma2:
            r.wait_recv()
        out_ref[...] = g2_ref[...].astype(jnp.float32)
        for r in rdma2:
            r.wait_send()

    out = pl.pallas_call(
        body,
        out_shape=jax.ShapeDtypeStruct((SQ, D), jnp.float32),
        in_specs=(
            [pl.BlockSpec(memory_space=pltpu.VMEM)] * 3
            + [pl.BlockSpec(memory_space=pl.ANY)] * 2
        ),
        out_specs=pl.BlockSpec(memory_space=pltpu.VMEM),
        scratch_shapes=[
            pltpu.VMEM((SQ, COMM_W), jnp.bfloat16),
            pltpu.VMEM((N_DEV, CHUNK, COMM_W), jnp.bfloat16),
            pltpu.VMEM((CHUNK, D), jnp.bfloat16),
            pltpu.VMEM((SQ, D), jnp.bfloat16),
            pltpu.VMEM((SKV_LOCAL, 2 * HKV * DH), jnp.float32),
            pltpu.SemaphoreType.DMA,
            pltpu.SemaphoreType.DMA,
            pltpu.SemaphoreType.DMA,
            pltpu.SemaphoreType.DMA,
            pltpu.SemaphoreType.DMA,
        ],
        compiler_params=pltpu.CompilerParams(collective_id=0),
    )(x2, Wq, Wo, k3, v3)
    return out.reshape(1, SQ, D)
